# Initial kernel scaffold; baseline (speedup 1.0000x reference)
import jax
import jax.numpy as jnp
from jax.experimental import pallas as pl


def kernel(user_h, item_h, W_user, W_item, Wg_u, Wg_i, by_src, by_dst, by_time, pby_src, pby_dst, pby_time):
    raise NotImplementedError("write your pallas kernel here")


# speedup vs baseline: 1.9913x; 1.9913x over previous
import functools

import jax
import jax.numpy as jnp
import numpy as np
from jax import lax
from jax.experimental import pallas as pl
from jax.experimental.pallas import tpu as pltpu
from jax.experimental.pallas import tpu_sc as plsc

_N = 10000
_E = 320000
_D = 128
_NW = 32
_NB = 320
_NPAD = _NW * _NB
_OFFS_LEN = _NPAD + 32
_SEGCAP = 4096
_WBUF = 8192
_INV_SCALE = 1.0 / float(np.sqrt(_D))


def _cape_table():
    r = np.arange(_SEGCAP, dtype=np.float32)[:, None]
    freqs = 1.0 / (10000.0 ** (np.arange(0, _D, 2, dtype=np.float32) / _D))
    theta = r * freqs[None, :]
    pe = np.zeros((_SEGCAP, _D), np.float32)
    pe[:, 0::2] = np.cos(theta)
    pe[:, 1::2] = np.sin(theta)
    return jnp.asarray(pe)


def _lin_kernel(x_ref, w_ref, o_ref):
    o_ref[...] = lax.dot_general(
        x_ref[...], w_ref[...], (((1,), (1,)), ((), ())),
        preferred_element_type=jnp.float32)


def _lin(x, W):
    n, d = x.shape
    br = 512
    return pl.pallas_call(
        _lin_kernel,
        grid=(pl.cdiv(n, br),),
        in_specs=[pl.BlockSpec((br, d), lambda i: (i, 0)),
                  pl.BlockSpec(W.shape, lambda i: (0, 0))],
        out_specs=pl.BlockSpec((br, W.shape[0]), lambda i: (i, 0)),
        out_shape=jax.ShapeDtypeStruct((n, W.shape[0]), jnp.float32),
    )(x, W)


def _post_kernel(h_ref, wg_ref, x_ref, o_ref):
    y = lax.dot_general(
        h_ref[...], wg_ref[...], (((1,), (1,)), ((), ())),
        preferred_element_type=jnp.float32) + x_ref[...]
    o_ref[...] = jnp.where(y > 0, y, jnp.exp(jnp.minimum(y, 0.0)) - 1.0)


def _post(h, Wg, x):
    n = x.shape[0]
    br = 512
    return pl.pallas_call(
        _post_kernel,
        grid=(pl.cdiv(n, br),),
        in_specs=[pl.BlockSpec((br, 2 * _D), lambda i: (i, 0)),
                  pl.BlockSpec(Wg.shape, lambda i: (0, 0)),
                  pl.BlockSpec((br, _D), lambda i: (i, 0))],
        out_specs=pl.BlockSpec((br, _D), lambda i: (i, 0)),
        out_shape=jax.ShapeDtypeStruct((n, _D), jnp.float32),
    )(h, Wg, x)


def _agg_sc(v_tab, q_tab, src_pad, time_pad, offs, cape):
    mesh = plsc.VectorSubcoreMesh(core_axis_name="c", subcore_axis_name="s")

    @functools.partial(
        pl.kernel,
        mesh=mesh,
        out_type=jax.ShapeDtypeStruct((_N, 2 * _D), jnp.float32),
        compiler_params=pltpu.CompilerParams(needs_layout_passes=False),
        scratch_types=[
            pltpu.VMEM((_NB + 32,), jnp.int32),
            pltpu.VMEM((_WBUF,), jnp.int32),
            pltpu.VMEM((_WBUF,), jnp.int32),
            pltpu.VMEM((_WBUF,), jnp.int32),
            pltpu.VMEM((_WBUF,), jnp.float32),
            pltpu.VMEM((_WBUF,), jnp.float32),
            pltpu.VMEM((1, _D), jnp.float32),
            pltpu.VMEM((1, _D), jnp.float32),
            pltpu.VMEM((16, _D), jnp.float32),
            pltpu.VMEM((16, _D), jnp.float32),
            pltpu.VMEM((1, 2 * _D), jnp.float32),
            pltpu.SemaphoreType.DMA,
            pltpu.SemaphoreType.DMA,
        ],
    )
    def agg(v_h, q_h, src_h, time_h, offs_h, cape_h, out_h,
            offs_v, swin, twin, rbuf, ebuf, e1buf, qrow, lastv, msgv, pev,
            outrow, sem0, sem1):
        cid = lax.axis_index("c")
        sid = lax.axis_index("s")
        w = sid * 2 + cid
        n0 = w * _NB
        pltpu.sync_copy(offs_h.at[pl.ds(n0, _NB + 32)], offs_v)
        lane = lax.iota(jnp.int32, 16)
        neg = jnp.float32(-1e30)

        def pick_i(vec, e):
            sel = jnp.where(lane == e, vec, 0).astype(jnp.float32)
            return jnp.sum(sel).astype(jnp.int32)

        def pick_f(vec, e):
            return jnp.sum(jnp.where(lane == e, vec, jnp.float32(0)))

        def node_chunk(ic, win_lo):
            ova = offs_v[pl.ds(ic * 16, 16)]
            ovb = offs_v[pl.ds(ic * 16 + 16, 16)]

            def node_body(ie, wl):
                i = ic * 16 + ie
                n = n0 + i
                s = pick_i(ova, ie)
                t = pick_i(ova, ie + 1) + pick_i(ovb, ie - 15)
                deg = jnp.minimum(t - s, _SEGCAP)
                t = s + deg
                new_win = (s // 8) * 8
                do_refill = (t - wl) > _WBUF

                @pl.when(do_refill)
                def _():
                    pltpu.sync_copy(src_h.at[pl.ds(new_win, _WBUF)], swin)
                    pltpu.sync_copy(time_h.at[pl.ds(new_win, _WBUF)], twin)

                wl = jnp.where(do_refill, new_win, wl)
                sloc = s - wl
                eloc = sloc + deg
                c0 = sloc // 16
                nch = jnp.where(deg > 0, (eloc + 15) // 16 - c0, 0)
                _node(n, sloc, eloc, c0, nch)
                return wl

            return lax.fori_loop(0, 16, node_body, win_lo)

        def _node(n, sloc, eloc, c0, nch):
            def rank_chunk(c, last_src):
                base = pl.multiple_of((c0 + c) * 16, 16)
                wpos = base + lane
                valid = (wpos >= sloc) & (wpos < eloc)
                tj = twin[pl.ds(base, 16)]

                def kc_body(kc, r):
                    kbase = pl.multiple_of((c0 + kc) * 16, 16)
                    tkv = twin[pl.ds(kbase, 16)]

                    def kk_body(kk, r2):
                        kpos = kbase + kk
                        tk = pick_i(tkv, kk)
                        okk = (kpos >= sloc) & (kpos < eloc)
                        gt = (tk > tj) | ((tk == tj) & (kpos > wpos))
                        return r2 + jnp.where(okk & gt, 1, 0)

                    return lax.fori_loop(0, 16, kk_body, r)

                r = lax.fori_loop(0, nch, kc_body,
                                  jnp.zeros((16,), jnp.int32))
                rbuf[pl.ds(base, 16)] = jnp.where(
                    valid, jnp.minimum(r, _SEGCAP - 1), _SEGCAP - 1)
                srcj = swin[pl.ds(base, 16)]
                hit = jnp.where((r == 0) & valid, srcj, 0)
                return last_src + jnp.sum(
                    hit.astype(jnp.float32)).astype(jnp.int32)

            last_src = lax.fori_loop(0, nch, rank_chunk, jnp.int32(0))

            pltpu.sync_copy(v_h.at[pl.ds(last_src, 1)], lastv)
            nq = jnp.minimum(n, _N - 1)
            pltpu.sync_copy(q_h.at[pl.ds(nq, 1)], qrow)
            qv = [qrow[0, pl.ds(dk * 16, 16)] for dk in range(8)]
            lv = [lastv[0, pl.ds(dk * 16, 16)] for dk in range(8)]

            def p1_chunk(c, mm):
                base = pl.multiple_of((c0 + c) * 16, 16)
                srcj = swin[pl.ds(base, 16)]
                wpos = base + lane
                valid = (wpos >= sloc) & (wpos < eloc)
                idx = jnp.where(valid, srcj, 0)
                cp1 = pltpu.async_copy(v_h.at[idx], msgv, sem0)
                rj = rbuf[pl.ds(base, 16)]
                cp2 = pltpu.async_copy(cape_h.at[rj], pev, sem1)
                cp1.wait()
                cp2.wait()

                def edge_body(e, vv):
                    evec, e1vec = vv
                    ea = jnp.zeros((16,), jnp.float32)
                    e1a = jnp.zeros((16,), jnp.float32)
                    for dk in range(8):
                        mg = msgv[e, pl.ds(dk * 16, 16)]
                        pe = pev[e, pl.ds(dk * 16, 16)]
                        ea = ea + (mg + pe) * qv[dk]
                        e1a = e1a + mg * lv[dk]
                    ej = jnp.sum(ea) * _INV_SCALE
                    e1j = jnp.sum(e1a) * _INV_SCALE
                    evec = jnp.where(lane == e, ej, evec)
                    e1vec = jnp.where(lane == e, e1j, e1vec)
                    return (evec, e1vec)

                evec, e1vec = lax.fori_loop(
                    0, 16, edge_body,
                    (jnp.full((16,), neg), jnp.full((16,), neg)))
                evec = jnp.where(valid, evec, neg)
                e1vec = jnp.where(valid, e1vec, neg)
                ebuf[pl.ds(base, 16)] = evec
                e1buf[pl.ds(base, 16)] = e1vec
                m_, m1_ = mm
                return (jnp.maximum(m_, jnp.max(evec)),
                        jnp.maximum(m1_, jnp.max(e1vec)))

            m, m1 = lax.fori_loop(0, nch, p1_chunk, (neg, neg))

            def z_chunk(c, zz):
                base = pl.multiple_of((c0 + c) * 16, 16)
                ex = jnp.exp(ebuf[pl.ds(base, 16)] - m)
                ex1 = jnp.exp(e1buf[pl.ds(base, 16)] - m1)
                return (zz[0] + jnp.sum(ex), zz[1] + jnp.sum(ex1))

            z, z1 = lax.fori_loop(0, nch, z_chunk,
                                  (jnp.float32(0), jnp.float32(0)))

            def p2_chunk(c, hh):
                base = pl.multiple_of((c0 + c) * 16, 16)
                srcj = swin[pl.ds(base, 16)]
                wpos = base + lane
                valid = (wpos >= sloc) & (wpos < eloc)
                idx = jnp.where(valid, srcj, 0)
                cp1 = pltpu.async_copy(v_h.at[idx], msgv, sem0)
                rj = rbuf[pl.ds(base, 16)]
                cp2 = pltpu.async_copy(cape_h.at[rj], pev, sem1)
                cp1.wait()
                cp2.wait()
                avec = jnp.exp(ebuf[pl.ds(base, 16)] - m) / (z + 1e-12)
                a1vec = jnp.exp(e1buf[pl.ds(base, 16)] - m1) / (z1 + 1e-12)
                avec = jnp.where(valid, avec, jnp.float32(0))
                a1vec = jnp.where(valid, a1vec, jnp.float32(0))

                def edge_body(e, hh2):
                    hl, hs = hh2
                    a = pick_f(avec, e)
                    a1 = pick_f(a1vec, e)
                    hl2 = []
                    hs2 = []
                    for dk in range(8):
                        mg = msgv[e, pl.ds(dk * 16, 16)]
                        pe = pev[e, pl.ds(dk * 16, 16)]
                        hl2.append(hl[dk] + a * (mg + pe))
                        hs2.append(hs[dk] + a1 * mg)
                    return (tuple(hl2), tuple(hs2))

                return lax.fori_loop(0, 16, edge_body, hh)

            zero8 = tuple(jnp.zeros((16,), jnp.float32) for _ in range(8))
            hl, hs = lax.fori_loop(0, nch, p2_chunk, (zero8, zero8))
            for dk in range(8):
                outrow[0, pl.ds(dk * 16, 16)] = hl[dk]
                outrow[0, pl.ds(_D + dk * 16, 16)] = hs[dk]

            @pl.when(n < _N)
            def _():
                pltpu.sync_copy(outrow, out_h.at[pl.ds(n, 1)])

        lax.fori_loop(0, _NB // 16, node_chunk, jnp.int32(-2 * _WBUF))

    return agg(v_tab, q_tab, src_pad, time_pad, offs, cape)


def _aggregate(q, v, src, time, offs, cape):
    src_pad = jnp.concatenate(
        [src.astype(jnp.int32), jnp.zeros((_WBUF,), jnp.int32)])
    time_pad = jnp.concatenate(
        [time.astype(jnp.int32), jnp.zeros((_WBUF,), jnp.int32)])
    return _agg_sc(v, q, src_pad, time_pad, offs, cape)


def kernel(user_h, item_h, W_user, W_item, Wg_u, Wg_i,
           by_src, by_dst, by_time, pby_src, pby_dst, pby_time):
    cape = _cape_table()
    queries = jnp.arange(_OFFS_LEN, dtype=jnp.int32)
    by_offs = jnp.searchsorted(by_dst, queries, side="left").astype(jnp.int32)
    pby_offs = jnp.searchsorted(pby_dst, queries,
                                side="left").astype(jnp.int32)
    u_t = _lin(user_h, W_user)
    i_t = _lin(item_h, W_item)
    h_u = _aggregate(u_t, i_t, by_src, by_time, by_offs, cape)
    h_i = _aggregate(i_t, u_t, pby_src, pby_time, pby_offs, cape)
    user_out = _post(h_u, Wg_u, user_h)
    item_out = _post(h_i, Wg_i, item_h)
    return user_out, item_out


# speedup vs baseline: 3.8329x; 1.9248x over previous
import functools

import jax
import jax.numpy as jnp
import numpy as np
from jax import lax
from jax.experimental import pallas as pl
from jax.experimental.pallas import tpu as pltpu
from jax.experimental.pallas import tpu_sc as plsc

_N = 10000
_E = 320000
_D = 128
_NW = 32
_NB = 320
_NPAD = _NW * _NB
_OFFS_LEN = _NPAD + 32
_SEGCAP = 4096
_WBUF = 8192
_INV_SCALE = 1.0 / float(np.sqrt(_D))


def _cape_table():
    r = np.arange(_SEGCAP, dtype=np.float32)[:, None]
    freqs = 1.0 / (10000.0 ** (np.arange(0, _D, 2, dtype=np.float32) / _D))
    theta = r * freqs[None, :]
    pe = np.zeros((_SEGCAP, _D), np.float32)
    pe[:, 0::2] = np.cos(theta)
    pe[:, 1::2] = np.sin(theta)
    return jnp.asarray(pe)


def _lin_kernel(x_ref, w_ref, o_ref):
    o_ref[...] = lax.dot_general(
        x_ref[...], w_ref[...], (((1,), (1,)), ((), ())),
        preferred_element_type=jnp.float32)


def _lin(x, W):
    n, d = x.shape
    br = 512
    return pl.pallas_call(
        _lin_kernel,
        grid=(pl.cdiv(n, br),),
        in_specs=[pl.BlockSpec((br, d), lambda i: (i, 0)),
                  pl.BlockSpec(W.shape, lambda i: (0, 0))],
        out_specs=pl.BlockSpec((br, W.shape[0]), lambda i: (i, 0)),
        out_shape=jax.ShapeDtypeStruct((n, W.shape[0]), jnp.float32),
    )(x, W)


def _post_kernel(h_ref, wg_ref, x_ref, o_ref):
    y = lax.dot_general(
        h_ref[...], wg_ref[...], (((1,), (1,)), ((), ())),
        preferred_element_type=jnp.float32) + x_ref[...]
    o_ref[...] = jnp.where(y > 0, y, jnp.exp(jnp.minimum(y, 0.0)) - 1.0)


def _post(h, Wg, x):
    n = x.shape[0]
    br = 512
    return pl.pallas_call(
        _post_kernel,
        grid=(pl.cdiv(n, br),),
        in_specs=[pl.BlockSpec((br, 2 * _D), lambda i: (i, 0)),
                  pl.BlockSpec(Wg.shape, lambda i: (0, 0)),
                  pl.BlockSpec((br, _D), lambda i: (i, 0))],
        out_specs=pl.BlockSpec((br, _D), lambda i: (i, 0)),
        out_shape=jax.ShapeDtypeStruct((n, _D), jnp.float32),
    )(h, Wg, x)


def _agg_sc(v_tab, q_tab, src_pad, time_pad, offs, cape):
    mesh = plsc.VectorSubcoreMesh(core_axis_name="c", subcore_axis_name="s")

    @functools.partial(
        pl.kernel,
        mesh=mesh,
        out_type=jax.ShapeDtypeStruct((_N, 2 * _D), jnp.float32),
        compiler_params=pltpu.CompilerParams(needs_layout_passes=False),
        scratch_types=[
            pltpu.VMEM((_NB + 32,), jnp.int32),
            pltpu.VMEM((_WBUF,), jnp.int32),
            pltpu.VMEM((_WBUF,), jnp.int32),
            pltpu.VMEM((_WBUF,), jnp.int32),
            pltpu.VMEM((1, _D), jnp.float32),
            pltpu.VMEM((1, _D), jnp.float32),
            pltpu.VMEM((16, _D), jnp.float32),
            pltpu.VMEM((16, _D), jnp.float32),
            pltpu.VMEM((16, _D), jnp.float32),
            pltpu.VMEM((16, _D), jnp.float32),
            pltpu.VMEM((1, 2 * _D), jnp.float32),
            pltpu.SemaphoreType.DMA,
            pltpu.SemaphoreType.DMA,
            pltpu.SemaphoreType.DMA,
            pltpu.SemaphoreType.DMA,
        ],
    )
    def agg(v_h, q_h, src_h, time_h, offs_h, cape_h, out_h,
            offs_v, swin, twin, rbuf, qrow, lastv, msgv0, msgv1, pev0,
            pev1, outrow, sem0, sem1, sem2, sem3):
        cid = lax.axis_index("c")
        sid = lax.axis_index("s")
        w = sid * 2 + cid
        n0 = w * _NB
        pltpu.sync_copy(offs_h.at[pl.ds(n0, _NB + 32)], offs_v)
        lane = lax.iota(jnp.int32, 16)
        neg = jnp.float32(-1e30)

        def _hsum(v):
            for sh in (8, 4, 2, 1):
                v = v + v[(lane + sh) % 16]
            return v

        def _hmax(v):
            for sh in (8, 4, 2, 1):
                v = jnp.maximum(v, v[(lane + sh) % 16])
            return v

        def pick_i(vec, e):
            return _hsum(jnp.where(lane == e, vec, 0))[0]

        def node_chunk(ic, win_lo):
            ova = offs_v[pl.ds(ic * 16, 16)]
            ovb = offs_v[pl.ds(ic * 16 + 16, 16)]

            def node_body(ie, wl):
                i = ic * 16 + ie
                n = n0 + i
                s = pick_i(ova, ie)
                t = pick_i(ova, ie + 1) + pick_i(ovb, ie - 15)
                deg = jnp.minimum(t - s, _SEGCAP)
                t = s + deg
                new_win = (s // 8) * 8
                do_refill = (t - wl) > _WBUF

                @pl.when(do_refill)
                def _():
                    pltpu.sync_copy(src_h.at[pl.ds(new_win, _WBUF)], swin)
                    pltpu.sync_copy(time_h.at[pl.ds(new_win, _WBUF)], twin)

                wl = jnp.where(do_refill, new_win, wl)
                sloc = s - wl
                eloc = sloc + deg
                c0 = sloc // 16
                nch = jnp.where(deg > 0, (eloc + 15) // 16 - c0, 0)
                _node(n, sloc, eloc, c0, nch)
                return wl

            return lax.fori_loop(0, 16, node_body, win_lo)

        def _node(n, sloc, eloc, c0, nch):
            nq = jnp.minimum(n, _N - 1)
            cpq = pltpu.async_copy(q_h.at[pl.ds(nq, 1)], qrow, sem2)

            def rank_chunk(c, last_carry):
                base = pl.multiple_of((c0 + c) * 16, 16)
                wpos = base + lane
                valid = (wpos >= sloc) & (wpos < eloc)
                tj = twin[pl.ds(base, 16)]

                def kc_body(kc, r):
                    kbase = pl.multiple_of((c0 + kc) * 16, 16)
                    tkv = twin[pl.ds(kbase, 16)]
                    for kk in range(16):
                        kpos = kbase + kk
                        tkb = _hsum(jnp.where(lane == kk, tkv, 0))
                        okk = (kpos >= sloc) & (kpos < eloc)
                        gt = (tkb > tj) | ((tkb == tj) & (kpos > wpos))
                        r = r + jnp.where(okk & gt, 1, 0)
                    return r

                r = lax.fori_loop(0, nch, kc_body,
                                  jnp.zeros((16,), jnp.int32))
                rbuf[pl.ds(base, 16)] = jnp.where(
                    valid, jnp.minimum(r, _SEGCAP - 1), _SEGCAP - 1)
                srcj = swin[pl.ds(base, 16)]
                hit = jnp.where((r == 0) & valid, srcj, 0)
                return last_carry + _hsum(hit)

            last_splat = lax.fori_loop(0, nch, rank_chunk,
                                       jnp.zeros((16,), jnp.int32))
            last_src = last_splat[0]

            def issue(c, msgb, peb):
                base = pl.multiple_of((c0 + c) * 16, 16)
                srcj = swin[pl.ds(base, 16)]
                wpos = base + lane
                valid = (wpos >= sloc) & (wpos < eloc)
                idx = jnp.where(valid, srcj, 0)
                pltpu.async_copy(v_h.at[idx], msgb, sem0)
                rj = rbuf[pl.ds(base, 16)]
                pltpu.async_copy(cape_h.at[rj], peb, sem1)

            @pl.when(nch > 0)
            def _():
                issue(0, msgv0, pev0)

            cpl = pltpu.async_copy(v_h.at[pl.ds(last_src, 1)], lastv, sem3)
            cpq.wait()
            cpl.wait()
            qv = [qrow[0, pl.ds(dk * 16, 16)] for dk in range(8)]
            lv = [lastv[0, pl.ds(dk * 16, 16)] for dk in range(8)]

            def half(c, msgb, peb, nmsgb, npeb, carry):
                proc = c < nch

                @pl.when(c + 1 < nch)
                def _():
                    issue(c + 1, nmsgb, npeb)

                @pl.when(proc)
                def _():
                    pltpu.make_async_copy(
                        v_h.at[pl.ds(0, 16)], msgb, sem0).wait()
                    pltpu.make_async_copy(
                        cape_h.at[pl.ds(0, 16)], peb, sem1).wait()

                m_, m1_, z_, z1_, hl_, hs_ = carry
                base = pl.multiple_of((c0 + c) * 16, 16)
                wpos = base + lane
                valid = (wpos >= sloc) & (wpos < eloc)

                def dot_body(e, vv):
                    evec, e1vec = vv
                    mg = [msgb[e, pl.ds(dk * 16, 16)] for dk in range(8)]
                    pe = [peb[e, pl.ds(dk * 16, 16)] for dk in range(8)]
                    ea0 = jnp.zeros((16,), jnp.float32)
                    ea1 = jnp.zeros((16,), jnp.float32)
                    f0 = jnp.zeros((16,), jnp.float32)
                    f1 = jnp.zeros((16,), jnp.float32)
                    for dk in range(0, 8, 2):
                        ea0 = ea0 + (mg[dk] + pe[dk]) * qv[dk]
                        ea1 = ea1 + (mg[dk + 1] + pe[dk + 1]) * qv[dk + 1]
                        f0 = f0 + mg[dk] * lv[dk]
                        f1 = f1 + mg[dk + 1] * lv[dk + 1]
                    ej = _hsum(ea0 + ea1) * _INV_SCALE
                    e1j = _hsum(f0 + f1) * _INV_SCALE
                    evec = jnp.where(lane == e, ej, evec)
                    e1vec = jnp.where(lane == e, e1j, e1vec)
                    return (evec, e1vec)

                evec, e1vec = lax.fori_loop(
                    0, 16, dot_body,
                    (jnp.full((16,), neg), jnp.full((16,), neg)))
                evec = jnp.where(valid, evec, neg)
                e1vec = jnp.where(valid, e1vec, neg)

                mn = jnp.maximum(m_, _hmax(evec))
                m1n = jnp.maximum(m1_, _hmax(e1vec))
                dv = jnp.exp(m_ - mn)
                dv1 = jnp.exp(m1_ - m1n)
                pv = jnp.where(valid, jnp.exp(evec - mn), jnp.float32(0))
                p1v = jnp.where(valid, jnp.exp(e1vec - m1n), jnp.float32(0))
                z_ = z_ * dv + _hsum(pv)
                z1_ = z1_ * dv1 + _hsum(p1v)
                hl_ = tuple(h * dv for h in hl_)
                hs_ = tuple(h * dv1 for h in hs_)

                def acc_body(e, hh):
                    hl2, hs2 = hh
                    a = _hsum(jnp.where(lane == e, pv, jnp.float32(0)))
                    a1 = _hsum(jnp.where(lane == e, p1v, jnp.float32(0)))
                    mg = [msgb[e, pl.ds(dk * 16, 16)] for dk in range(8)]
                    pe = [peb[e, pl.ds(dk * 16, 16)] for dk in range(8)]
                    hl2 = tuple(hl2[dk] + a * (mg[dk] + pe[dk])
                                for dk in range(8))
                    hs2 = tuple(hs2[dk] + a1 * mg[dk] for dk in range(8))
                    return (hl2, hs2)

                hl_, hs_ = lax.fori_loop(0, 16, acc_body, (hl_, hs_))
                new = (mn, m1n, z_, z1_, hl_, hs_)
                return jax.tree.map(
                    lambda a, b: jnp.where(proc, a, b), new, carry)

            def pair_body(g, carry):
                c = 2 * g
                carry = half(c, msgv0, pev0, msgv1, pev1, carry)
                carry = half(c + 1, msgv1, pev1, msgv0, pev0, carry)
                return carry

            zf = jnp.zeros((16,), jnp.float32)
            zero8 = tuple(zf for _ in range(8))
            negv = jnp.full((16,), neg)
            m, m1, z, z1, hl, hs = lax.fori_loop(
                0, (nch + 1) // 2, pair_body,
                (negv, negv, zf, zf, zero8, zero8))
            for dk in range(8):
                outrow[0, pl.ds(dk * 16, 16)] = hl[dk] / (z + 1e-12)
                outrow[0, pl.ds(_D + dk * 16, 16)] = hs[dk] / (z1 + 1e-12)

            @pl.when(n < _N)
            def _():
                pltpu.sync_copy(outrow, out_h.at[pl.ds(n, 1)])

        lax.fori_loop(0, _NB // 16, node_chunk, jnp.int32(-2 * _WBUF))

    return agg(v_tab, q_tab, src_pad, time_pad, offs, cape)


def _aggregate(q, v, src, time, offs, cape):
    src_pad = jnp.concatenate(
        [src.astype(jnp.int32), jnp.zeros((_WBUF,), jnp.int32)])
    time_pad = jnp.concatenate(
        [time.astype(jnp.int32), jnp.zeros((_WBUF,), jnp.int32)])
    return _agg_sc(v, q, src_pad, time_pad, offs, cape)


def kernel(user_h, item_h, W_user, W_item, Wg_u, Wg_i,
           by_src, by_dst, by_time, pby_src, pby_dst, pby_time):
    cape = _cape_table()
    queries = jnp.arange(_OFFS_LEN, dtype=jnp.int32)
    by_offs = jnp.searchsorted(by_dst, queries, side="left").astype(jnp.int32)
    pby_offs = jnp.searchsorted(pby_dst, queries,
                                side="left").astype(jnp.int32)
    u_t = _lin(user_h, W_user)
    i_t = _lin(item_h, W_item)
    h_u = _aggregate(u_t, i_t, by_src, by_time, by_offs, cape)
    h_i = _aggregate(i_t, u_t, pby_src, pby_time, pby_offs, cape)
    user_out = _post(h_u, Wg_u, user_h)
    item_out = _post(h_i, Wg_i, item_h)
    return user_out, item_out
